# baseline (device time: 8675 ns/iter reference)
import jax
import jax.numpy as jnp
from jax import lax
from jax.experimental import pallas as pl
from jax.experimental.pallas import tpu as pltpu

N_DEV = 4
N_HALF = 2
EPS = 1e-5


def kernel(x, gamma, beta):
    m, n_per = x.shape
    n_global = n_per * N_DEV
    mh = m // N_HALF
    assert mh % 128 == 0
    mrh = mh // 128

    import os
    _scopes = os.environ.get("KERNEL_SCOPES", "0") == "1"

    class _noscope:
        def __init__(self, name):
            self._cm = jax.named_scope(name) if _scopes else None

        def __enter__(self):
            if self._cm:
                self._cm.__enter__()

        def __exit__(self, *a):
            if self._cm:
                self._cm.__exit__(*a)

    def body(x_ref, gb_ref, out_ref,
             mystats_ref, comm_ref, gxf_ref,
             send_sems, recv_sems):
        my = lax.axis_index("i")

        with _noscope("phase_signal"):
            barrier = pltpu.get_barrier_semaphore()
            for k in range(1, N_DEV):
                peer = lax.rem(my + k, N_DEV)
                pl.semaphore_signal(
                    barrier, inc=1,
                    device_id=(peer,), device_id_type=pl.DeviceIdType.MESH,
                )

        def stats_half(h):
            x3 = x_ref[h * mh:(h + 1) * mh, :].reshape(mrh, 128, n_per)
            mystats_ref[h, 0:mrh] = jnp.sum(x3, axis=2)
            mystats_ref[h, mrh:2 * mrh] = jnp.sum(x3 * x3, axis=2)

        def send_half(h):
            rdmas = []
            for k in range(1, N_DEV):
                peer = lax.rem(my + k, N_DEV)
                rdma = pltpu.make_async_remote_copy(
                    src_ref=mystats_ref.at[h],
                    dst_ref=comm_ref.at[k - 1, h],
                    send_sem=send_sems.at[k - 1, h],
                    recv_sem=recv_sems.at[k - 1, h],
                    device_id=(peer,),
                    device_id_type=pl.DeviceIdType.MESH,
                )
                rdma.start()
                rdmas.append(rdma)
            return rdmas

        with _noscope("phase_stats0"):
            stats_half(0)
        with _noscope("phase_barrier_wait"):
            pl.semaphore_wait(barrier, N_DEV - 1)
        with _noscope("phase_send0"):
            rdmas0 = send_half(0)
        with _noscope("phase_stats1"):
            stats_half(1)
        with _noscope("phase_send1"):
            rdmas1 = send_half(1)

        with _noscope("phase_gxf"):
            g_row = gb_ref[0:n_per][None, :]
            b_row = gb_ref[n_per:2 * n_per][None, :]
            gxf_ref[...] = (g_row * x_ref[...]).astype(jnp.bfloat16)
        g3 = g_row[None]
        b3 = b_row[None]

        def normalize_half(h, rdmas):
            with _noscope(f"phase_wait_recv{h}"):
                for rdma in rdmas:
                    rdma.wait_recv()
            with _noscope(f"phase_normalize{h}"):
                total = mystats_ref[h]
                for k in range(N_DEV - 1):
                    total = total + comm_ref[k, h]
                mean = total[0:mrh, :, None] * (1.0 / n_global)
                ex2 = total[mrh:2 * mrh, :, None] * (1.0 / n_global)
                var = ex2 - mean * mean
                inv = lax.rsqrt(var + EPS)
                rows = slice(h * mh, (h + 1) * mh)
                gxf3 = gxf_ref[rows].reshape(mrh, 128, n_per)
                out3 = gxf3 * inv - g3 * (mean * inv) + b3
                out_ref[rows] = out3.reshape(mh, n_per).astype(out_ref.dtype)

        normalize_half(0, rdmas0)
        normalize_half(1, rdmas1)

        with _noscope("phase_wait_send"):
            for rdma in rdmas0 + rdmas1:
                rdma.wait_send()

    return pl.pallas_call(
        body,
        out_shape=jax.ShapeDtypeStruct((m, n_per), jnp.bfloat16),
        in_specs=[pl.BlockSpec(memory_space=pltpu.VMEM)] * 2,
        out_specs=pl.BlockSpec(memory_space=pltpu.VMEM),
        scratch_shapes=[
            pltpu.VMEM((N_HALF, 2 * mrh, 128), jnp.float32),
            pltpu.VMEM((N_DEV - 1, N_HALF, 2 * mrh, 128), jnp.float32),
            pltpu.VMEM((m, n_per), jnp.bfloat16),
            pltpu.SemaphoreType.DMA((N_DEV - 1, N_HALF)),
            pltpu.SemaphoreType.DMA((N_DEV - 1, N_HALF)),
        ],
        compiler_params=pltpu.CompilerParams(collective_id=0),
    )(x, jnp.concatenate([gamma, beta]))
